# baseline (speedup 1.0000x reference)
import functools

import jax
import jax.numpy as jnp
from jax import lax
from jax.experimental import pallas as pl
from jax.experimental.pallas import tpu as pltpu
from jax.experimental.pallas import tpu_sc as plsc

_EMB_NUM = 1000000
_EMB_DIM = 64
_BATCH = 16384
_TEXT_LEN = 200
_TAG_LEN = 20

_BN = 16384
_GRID_A = -(-_EMB_NUM // _BN)
_NPAD = _GRID_A * _BN

_LANES = 16


def _precompute_body(wct_ref, embt_ref, p1_ref, p2_ref):
    r = lax.dot_general(wct_ref[...], embt_ref[...],
                        dimension_numbers=(((1,), (0,)), ((), ())),
                        preferred_element_type=jnp.float32)
    p1_ref[...] = r[0, :]
    p2_ref[...] = r[1, :]


def _precompute(embt, wct):
    p1, p2 = pl.pallas_call(
        _precompute_body,
        grid=(_GRID_A,),
        in_specs=[
            pl.BlockSpec((2, _EMB_DIM), lambda k: (0, 0)),
            pl.BlockSpec((_EMB_DIM, _BN), lambda k: (0, k)),
        ],
        out_specs=[
            pl.BlockSpec((_BN,), lambda k: (k,)),
            pl.BlockSpec((_BN,), lambda k: (k,)),
        ],
        out_shape=[
            jax.ShapeDtypeStruct((_NPAD,), jnp.float32),
            jax.ShapeDtypeStruct((_NPAD,), jnp.float32),
        ],
    )(wct, embt)
    return p1, p2


def _tree_sum(parts):
    while len(parts) > 1:
        nxt = [a + b for a, b in zip(parts[::2], parts[1::2])]
        if len(parts) % 2:
            nxt.append(parts[-1])
        parts = nxt
    return parts[0]


def _make_sc_lookup(nw):
    rows_w = _BATCH // nw
    groups_w = rows_w // _LANES
    txt_dmas = _TEXT_LEN * _LANES // 128
    tag_rows_w = rows_w * _TAG_LEN // 128

    mesh = plsc.VectorSubcoreMesh(core_axis_name="c", subcore_axis_name="s")

    def _sum_group(val_ref):
        parts = []
        for j in range(_TEXT_LEN):
            off = j * _LANES
            parts.append(val_ref[off // 128, pl.ds(off % 128, _LANES)])
        return _tree_sum(parts)

    @functools.partial(
        pl.kernel,
        out_type=jax.ShapeDtypeStruct((_BATCH,), jnp.float32),
        mesh=mesh,
        scratch_types=[
            pltpu.VMEM((tag_rows_w, 128), jnp.int32),
            pltpu.VMEM((tag_rows_w, 128), jnp.float32),
            pltpu.VMEM((txt_dmas, 128), jnp.int32),
            pltpu.VMEM((txt_dmas, 128), jnp.float32),
            pltpu.VMEM((txt_dmas, 128), jnp.int32),
            pltpu.VMEM((txt_dmas, 128), jnp.float32),
            pltpu.VMEM((rows_w,), jnp.float32),
            pltpu.VMEM((_LANES,), jnp.float32),
            pltpu.SemaphoreType.DMA,
            pltpu.SemaphoreType.DMA,
            pltpu.SemaphoreType.DMA,
            pltpu.SemaphoreType.DMA,
            pltpu.SemaphoreType.DMA,
        ],
    )
    def sc_lookup(p1_hbm, p2_hbm, textT_hbm, tagT_hbm, b_hbm, out_hbm,
                  idx_tag, val_tag, idx_a, val_a, idx_b, val_b, acc, b_v,
                  sem_tag, sem_ga, sem_gb, sem_ia, sem_ib):
        wid = lax.axis_index("s") * 2 + lax.axis_index("c")
        base = wid * groups_w
        pltpu.sync_copy(b_hbm, b_v)

        pltpu.sync_copy(tagT_hbm.at[pl.ds(wid * tag_rows_w, tag_rows_w)],
                        idx_tag)

        def fire_tag(j, c):
            pltpu.async_copy(p2_hbm.at[idx_tag.at[j]], val_tag.at[j],
                             sem_tag)
            return c
        lax.fori_loop(0, tag_rows_w, fire_tag, 0)

        def fire(idx_ref, val_ref, sem, j, c):
            pltpu.async_copy(p1_hbm.at[idx_ref.at[j]], val_ref.at[j], sem)
            return c

        def drain(idx_ref, val_ref, sem, j, c):
            pltpu.make_async_copy(p1_hbm.at[idx_ref.at[j]], val_ref.at[j],
                                  sem).wait()
            return c

        pltpu.async_copy(textT_hbm.at[base + 0], idx_a, sem_ia)
        pltpu.async_copy(textT_hbm.at[base + 1], idx_b, sem_ib)
        pltpu.make_async_copy(textT_hbm.at[base + 0], idx_a, sem_ia).wait()
        lax.fori_loop(0, txt_dmas,
                      functools.partial(fire, idx_a, val_a, sem_ga), 0)

        def pair(t, c):
            g0 = 2 * t
            pltpu.make_async_copy(textT_hbm.at[base + g0 + 1], idx_b,
                                  sem_ib).wait()
            lax.fori_loop(0, txt_dmas,
                          functools.partial(fire, idx_b, val_b, sem_gb), 0)
            lax.fori_loop(0, txt_dmas,
                          functools.partial(drain, idx_a, val_a, sem_ga), 0)

            @pl.when(t < groups_w // 2 - 1)
            def _():
                pltpu.async_copy(textT_hbm.at[base + g0 + 2], idx_a, sem_ia)
            acc[pl.ds(g0 * _LANES, _LANES)] = _sum_group(val_a)

            @pl.when(t < groups_w // 2 - 1)
            def _():
                pltpu.make_async_copy(textT_hbm.at[base + g0 + 2], idx_a,
                                      sem_ia).wait()
                lax.fori_loop(0, txt_dmas,
                              functools.partial(fire, idx_a, val_a, sem_ga),
                              0)
            lax.fori_loop(0, txt_dmas,
                          functools.partial(drain, idx_b, val_b, sem_gb), 0)
            acc[pl.ds((g0 + 1) * _LANES, _LANES)] = _sum_group(val_b)

            @pl.when(t < groups_w // 2 - 1)
            def _():
                pltpu.async_copy(textT_hbm.at[base + g0 + 3], idx_b, sem_ib)
            return c
        lax.fori_loop(0, groups_w // 2, pair, 0)

        def drain_tag(j, c):
            pltpu.make_async_copy(p2_hbm.at[idx_tag.at[j]], val_tag.at[j],
                                  sem_tag).wait()
            return c
        lax.fori_loop(0, tag_rows_w, drain_tag, 0)

        for g in range(groups_w):
            parts = []
            for j in range(_TAG_LEN):
                off = (g * _TAG_LEN + j) * _LANES
                parts.append(val_tag[off // 128, pl.ds(off % 128, _LANES)])
            s_tag = _tree_sum(parts)
            s_txt = acc[pl.ds(g * _LANES, _LANES)]
            x = (s_txt * (1.0 / _TEXT_LEN) + s_tag * (1.0 / _TAG_LEN)
                 + b_v[...])
            acc[pl.ds(g * _LANES, _LANES)] = 1.0 / (1.0 + jnp.exp(-x))

        pltpu.sync_copy(acc, out_hbm.at[pl.ds(wid * rows_w, rows_w)])

    return sc_lookup


def kernel(text, tag, text_length, emb_table, W, b):
    del text_length

    info = plsc.get_sparse_core_info()
    nw = info.num_cores * info.num_subcores

    wct = W.reshape(2, _EMB_DIM)
    p1, p2 = _precompute(emb_table.T, wct)

    ng = _BATCH // _LANES
    textT = (text.reshape(ng, _LANES, _TEXT_LEN).transpose(0, 2, 1)
             .reshape(ng, _TEXT_LEN * _LANES // 128, 128))
    tagT = (tag.reshape(ng, _LANES, _TAG_LEN).transpose(0, 2, 1)
            .reshape(ng * _TAG_LEN * _LANES // 128, 128))
    b16 = jnp.broadcast_to(b, (_LANES,))

    out = _make_sc_lookup(nw)(p1, p2, textT, tagT, b16)
    return out.reshape(_BATCH, 1)


# speedup vs baseline: 1.0311x; 1.0311x over previous
import functools

import jax
import jax.numpy as jnp
from jax import lax
from jax.experimental import pallas as pl
from jax.experimental.pallas import tpu as pltpu
from jax.experimental.pallas import tpu_sc as plsc

_EMB_NUM = 1000000
_EMB_DIM = 64
_BATCH = 16384
_TEXT_LEN = 200
_TAG_LEN = 20

_BN = 32768
_GRID_A = -(-_EMB_NUM // _BN)
_NPAD = _GRID_A * _BN

_LANES = 16


def _precompute_body(wct_ref, embt_ref, p1_ref, p2_ref):
    r = lax.dot_general(wct_ref[...], embt_ref[...],
                        dimension_numbers=(((1,), (0,)), ((), ())),
                        preferred_element_type=jnp.float32)
    p1_ref[...] = r[0, :]
    p2_ref[...] = r[1, :]


def _precompute(embt, wct):
    p1, p2 = pl.pallas_call(
        _precompute_body,
        grid=(_GRID_A,),
        in_specs=[
            pl.BlockSpec((2, _EMB_DIM), lambda k: (0, 0)),
            pl.BlockSpec((_EMB_DIM, _BN), lambda k: (0, k)),
        ],
        out_specs=[
            pl.BlockSpec((_BN,), lambda k: (k,)),
            pl.BlockSpec((_BN,), lambda k: (k,)),
        ],
        out_shape=[
            jax.ShapeDtypeStruct((_NPAD,), jnp.float32),
            jax.ShapeDtypeStruct((_NPAD,), jnp.float32),
        ],
    )(wct, embt)
    return p1, p2


def _tree_sum(parts):
    while len(parts) > 1:
        nxt = [a + b for a, b in zip(parts[::2], parts[1::2])]
        if len(parts) % 2:
            nxt.append(parts[-1])
        parts = nxt
    return parts[0]


def _make_sc_lookup(nw):
    rows_w = _BATCH // nw
    groups_w = rows_w // _LANES
    txt_dmas = _TEXT_LEN * _LANES // 128
    tag_rows_w = rows_w * _TAG_LEN // 128

    mesh = plsc.VectorSubcoreMesh(core_axis_name="c", subcore_axis_name="s")

    def _sum_group(val_ref):
        parts = []
        for j in range(_TEXT_LEN):
            off = j * _LANES
            parts.append(val_ref[off // 128, pl.ds(off % 128, _LANES)])
        return _tree_sum(parts)

    @functools.partial(
        pl.kernel,
        out_type=jax.ShapeDtypeStruct((_BATCH,), jnp.float32),
        mesh=mesh,
        scratch_types=[
            pltpu.VMEM((tag_rows_w, 128), jnp.int32),
            pltpu.VMEM((tag_rows_w, 128), jnp.float32),
            pltpu.VMEM((txt_dmas, 128), jnp.int32),
            pltpu.VMEM((txt_dmas, 128), jnp.float32),
            pltpu.VMEM((txt_dmas, 128), jnp.int32),
            pltpu.VMEM((txt_dmas, 128), jnp.float32),
            pltpu.VMEM((rows_w,), jnp.float32),
            pltpu.VMEM((_LANES,), jnp.float32),
            pltpu.SemaphoreType.DMA,
            pltpu.SemaphoreType.DMA,
            pltpu.SemaphoreType.DMA,
            pltpu.SemaphoreType.DMA,
            pltpu.SemaphoreType.DMA,
        ],
    )
    def sc_lookup(p1_hbm, p2_hbm, textT_hbm, tagT_hbm, b_hbm, out_hbm,
                  idx_tag, val_tag, idx_a, val_a, idx_b, val_b, acc, b_v,
                  sem_tag, sem_ga, sem_gb, sem_ia, sem_ib):
        wid = lax.axis_index("s") * 2 + lax.axis_index("c")
        base = wid * groups_w
        pltpu.sync_copy(b_hbm, b_v)

        pltpu.sync_copy(tagT_hbm.at[pl.ds(wid * tag_rows_w, tag_rows_w)],
                        idx_tag)

        def fire_tag(j, c):
            pltpu.async_copy(p2_hbm.at[idx_tag.at[j]], val_tag.at[j],
                             sem_tag)
            return c
        lax.fori_loop(0, tag_rows_w, fire_tag, 0)

        def fire(idx_ref, val_ref, sem, j, c):
            pltpu.async_copy(p1_hbm.at[idx_ref.at[j]], val_ref.at[j], sem)
            return c

        def drain(idx_ref, val_ref, sem, j, c):
            pltpu.make_async_copy(p1_hbm.at[idx_ref.at[j]], val_ref.at[j],
                                  sem).wait()
            return c

        pltpu.async_copy(textT_hbm.at[base + 0], idx_a, sem_ia)
        pltpu.async_copy(textT_hbm.at[base + 1], idx_b, sem_ib)
        pltpu.make_async_copy(textT_hbm.at[base + 0], idx_a, sem_ia).wait()
        lax.fori_loop(0, txt_dmas,
                      functools.partial(fire, idx_a, val_a, sem_ga), 0)

        def pair(t, c):
            g0 = 2 * t
            pltpu.make_async_copy(textT_hbm.at[base + g0 + 1], idx_b,
                                  sem_ib).wait()
            lax.fori_loop(0, txt_dmas,
                          functools.partial(fire, idx_b, val_b, sem_gb), 0)
            lax.fori_loop(0, txt_dmas,
                          functools.partial(drain, idx_a, val_a, sem_ga), 0)

            @pl.when(t < groups_w // 2 - 1)
            def _():
                pltpu.async_copy(textT_hbm.at[base + g0 + 2], idx_a, sem_ia)
            acc[pl.ds(g0 * _LANES, _LANES)] = _sum_group(val_a)

            @pl.when(t < groups_w // 2 - 1)
            def _():
                pltpu.make_async_copy(textT_hbm.at[base + g0 + 2], idx_a,
                                      sem_ia).wait()
                lax.fori_loop(0, txt_dmas,
                              functools.partial(fire, idx_a, val_a, sem_ga),
                              0)
            lax.fori_loop(0, txt_dmas,
                          functools.partial(drain, idx_b, val_b, sem_gb), 0)

            @pl.when(t < groups_w // 2 - 1)
            def _():
                pltpu.async_copy(textT_hbm.at[base + g0 + 3], idx_b, sem_ib)
            acc[pl.ds((g0 + 1) * _LANES, _LANES)] = _sum_group(val_b)
            return c
        lax.fori_loop(0, groups_w // 2, pair, 0)

        def drain_tag(j, c):
            pltpu.make_async_copy(p2_hbm.at[idx_tag.at[j]], val_tag.at[j],
                                  sem_tag).wait()
            return c
        lax.fori_loop(0, tag_rows_w, drain_tag, 0)

        for g in range(groups_w):
            parts = []
            for j in range(_TAG_LEN):
                off = (g * _TAG_LEN + j) * _LANES
                parts.append(val_tag[off // 128, pl.ds(off % 128, _LANES)])
            s_tag = _tree_sum(parts)
            s_txt = acc[pl.ds(g * _LANES, _LANES)]
            x = (s_txt * (1.0 / _TEXT_LEN) + s_tag * (1.0 / _TAG_LEN)
                 + b_v[...])
            acc[pl.ds(g * _LANES, _LANES)] = 1.0 / (1.0 + jnp.exp(-x))

        pltpu.sync_copy(acc, out_hbm.at[pl.ds(wid * rows_w, rows_w)])

    return sc_lookup


def kernel(text, tag, text_length, emb_table, W, b):
    del text_length

    info = plsc.get_sparse_core_info()
    nw = info.num_cores * info.num_subcores

    wct = W.reshape(2, _EMB_DIM)
    p1, p2 = _precompute(emb_table.T, wct)

    ng = _BATCH // _LANES
    textT = (text.reshape(ng, _LANES, _TEXT_LEN).transpose(0, 2, 1)
             .reshape(ng, _TEXT_LEN * _LANES // 128, 128))
    tagT = (tag.reshape(ng, _LANES, _TAG_LEN).transpose(0, 2, 1)
            .reshape(ng * _TAG_LEN * _LANES // 128, 128))
    b16 = jnp.broadcast_to(b, (_LANES,))

    out = _make_sc_lookup(nw)(p1, p2, textT, tagT, b16)
    return out.reshape(_BATCH, 1)
